# baseline (speedup 1.0000x reference)
import jax
import jax.numpy as jnp
from jax.experimental import pallas as pl

_B, _T, _C, _H, _W = 8, 16, 64, 64, 64
_T2 = _T // 2
_Cn = 16
_CH = 16


def _body(x_ref, v_ref, o_ref):
    o_ref[...] = x_ref[...]
    lane = jax.lax.broadcasted_iota(jnp.int32, (32, 128), 1)
    even = (lane % 2) == 0
    for k in range(_CH // 4):
        x0 = x_ref[0, 0, 4 * k]
        v = v_ref[0, 0, k]
        vexp = jnp.broadcast_to(v[:, :, None], (32, 64, 2)).reshape(32, 128)
        o_ref[0, 0, 4 * k] = jnp.where(even, vexp, x0)


def kernel(x_aug, values):
    x5 = x_aug.reshape(_B, _T, _C, 32, 128)
    v5 = values.reshape(_B, _T2, _Cn, 32, 64)
    out = pl.pallas_call(
        _body,
        grid=(_B, _T2, _C // _CH),
        in_specs=[
            pl.BlockSpec((1, 2, _CH, 32, 128), lambda b, j, g: (b, j, g, 0, 0)),
            pl.BlockSpec((1, 1, _CH // 4, 32, 64), lambda b, j, g: (b, j, g, 0, 0)),
        ],
        out_specs=pl.BlockSpec((1, 2, _CH, 32, 128), lambda b, j, g: (b, j, g, 0, 0)),
        out_shape=jax.ShapeDtypeStruct((_B, _T, _C, 32, 128), jnp.float32),
    )(x5, v5)
    return out.reshape(_B, _T, _C, _H, _W)


# speedup vs baseline: 1.6548x; 1.6548x over previous
import jax
import jax.numpy as jnp
from jax.experimental import pallas as pl

_B, _T, _C, _H, _W = 8, 16, 64, 64, 64
_T2 = _T // 2
_Cn = 16
_CH = 16


def _body(x_ref, v_ref, o_ref):
    o_ref[...] = x_ref[...]
    lane = jax.lax.broadcasted_iota(jnp.int32, (32, 128), 1)
    even = (lane % 2) == 0
    row = jax.lax.broadcasted_iota(jnp.int32, (64, 128), 0)
    col = jax.lax.broadcasted_iota(jnp.int32, (64, 128), 1)
    expand = jnp.where(col // 2 == row, 1.0, 0.0).astype(jnp.float32)
    for k in range(_CH // 4):
        x0 = x_ref[0, 0, 4 * k]
        v = v_ref[0, 0, k]
        vexp = jax.lax.dot_general(
            v, expand, (((1,), (0,)), ((), ())),
            preferred_element_type=jnp.float32,
            precision=jax.lax.Precision.HIGHEST)
        o_ref[0, 0, 4 * k] = jnp.where(even, vexp, x0)


def kernel(x_aug, values):
    x5 = x_aug.reshape(_B, _T, _C, 32, 128)
    v5 = values.reshape(_B, _T2, _Cn, 32, 64)
    out = pl.pallas_call(
        _body,
        grid=(_B, _T2, _C // _CH),
        in_specs=[
            pl.BlockSpec((1, 2, _CH, 32, 128), lambda b, j, g: (b, j, g, 0, 0)),
            pl.BlockSpec((1, 1, _CH // 4, 32, 64), lambda b, j, g: (b, j, g, 0, 0)),
        ],
        out_specs=pl.BlockSpec((1, 2, _CH, 32, 128), lambda b, j, g: (b, j, g, 0, 0)),
        out_shape=jax.ShapeDtypeStruct((_B, _T, _C, 32, 128), jnp.float32),
    )(x5, v5)
    return out.reshape(_B, _T, _C, _H, _W)


# speedup vs baseline: 1.9924x; 1.2040x over previous
import jax
import jax.numpy as jnp
from jax.experimental import pallas as pl

_B, _T, _C, _H, _W = 8, 16, 64, 64, 64
_T2 = _T // 2
_Cn = 16
_CH = 64


def _body(x_ref, v_ref, o_ref):
    o_ref[...] = x_ref[...]
    lane = jax.lax.broadcasted_iota(jnp.int32, (32, 128), 1)
    even = (lane % 2) == 0
    row = jax.lax.broadcasted_iota(jnp.int32, (64, 128), 0)
    col = jax.lax.broadcasted_iota(jnp.int32, (64, 128), 1)
    expand = jnp.where(col // 2 == row, 1.0, 0.0).astype(jnp.float32)
    for k in range(_CH // 4):
        x0 = x_ref[0, 0, 4 * k]
        v = v_ref[0, 0, k]
        vexp = jax.lax.dot_general(
            v, expand, (((1,), (0,)), ((), ())),
            preferred_element_type=jnp.float32,
            precision=jax.lax.Precision.HIGHEST)
        o_ref[0, 0, 4 * k] = jnp.where(even, vexp, x0)


def kernel(x_aug, values):
    x5 = x_aug.reshape(_B, _T, _C, 32, 128)
    v5 = values.reshape(_B, _T2, _Cn, 32, 64)
    out = pl.pallas_call(
        _body,
        grid=(_B, _T2, _C // _CH),
        in_specs=[
            pl.BlockSpec((1, 2, _CH, 32, 128), lambda b, j, g: (b, j, g, 0, 0)),
            pl.BlockSpec((1, 1, _CH // 4, 32, 64), lambda b, j, g: (b, j, g, 0, 0)),
        ],
        out_specs=pl.BlockSpec((1, 2, _CH, 32, 128), lambda b, j, g: (b, j, g, 0, 0)),
        out_shape=jax.ShapeDtypeStruct((_B, _T, _C, 32, 128), jnp.float32),
    )(x5, v5)
    return out.reshape(_B, _T, _C, _H, _W)
